# Initial kernel scaffold; baseline (speedup 1.0000x reference)
import jax
import jax.numpy as jnp
from jax.experimental import pallas as pl


def kernel(logits, targets):
    raise NotImplementedError("write your pallas kernel here")


# speedup vs baseline: 1.6799x; 1.6799x over previous
import functools

import jax
import jax.numpy as jnp
from jax import lax
from jax.experimental import pallas as pl
from jax.experimental.pallas import tpu as pltpu
from jax.experimental.pallas import tpu_sc as plsc

B = 128
N = 32768
K = 12
L = 16
NW = 32
ROWS_PER_W = B // NW
UNROLL = 8
NBLK = N // (L * UNROLL)
NEG = -3.0e38


def _lane(x, k):
    i = lax.iota(jnp.int32, L)
    return jnp.max(jnp.where(i == k, x, NEG))


def _merge(v, base, cand_v, cand_i):
    idx = base + lax.iota(jnp.int32, L)
    sv, si = plsc.sort_key_val(v, idx, descending=True)
    rcv = lax.rev(cand_v, (0,))
    rci = lax.rev(cand_i, (0,))
    take = sv > rcv
    nv = jnp.where(take, sv, rcv)
    ni = jnp.where(take, si, rci)
    cv2, ci2 = plsc.sort_key_val(nv, ni, descending=True)
    return cv2, ci2, _lane(cv2, K - 1)


def _sc_body(logits_hbm, targets_hbm, tab_hbm, out_hbm, log_v, tgt_v, tab_v,
             res_v):
    wid = lax.axis_index("c") * 16 + lax.axis_index("s")
    iota = lax.iota(jnp.int32, L)
    pltpu.sync_copy(tab_hbm, tab_v)
    inv_ranks = tab_v[0]
    rec_denom = tab_v[1]
    mask12 = (iota < K).astype(jnp.float32)

    for r_local in range(ROWS_PER_W):
        row = wid * ROWS_PER_W + r_local
        pltpu.sync_copy(logits_hbm.at[row], log_v)
        pltpu.sync_copy(targets_hbm.at[row], tgt_v)

        def sum_body(b, acc):
            base = b * (L * UNROLL)
            for u in range(UNROLL):
                acc = acc + tgt_v[pl.ds(base + u * L, L)]
            return acc

        acc = lax.fori_loop(0, NBLK, sum_body, jnp.zeros((L,), jnp.int32))
        tsum = jnp.sum(acc)

        def blk_body(b, carry):
            cand_v, cand_i, t = carry
            base = b * (L * UNROLL)
            chunks = [log_v[pl.ds(base + u * L, L)] for u in range(UNROLL)]
            mm = chunks[0]
            for u in range(1, UNROLL):
                mm = jnp.maximum(mm, chunks[u])
            bmax = jnp.max(mm)

            def slow(carry):
                for u in range(UNROLL):
                    cv, ci, tt = carry
                    cmax = jnp.max(chunks[u])
                    carry = lax.cond(
                        cmax > tt,
                        lambda cv=cv, ci=ci, u=u: _merge(
                            chunks[u], base + u * L, cv, ci),
                        lambda cv=cv, ci=ci, tt=tt: (cv, ci, tt),
                    )
                return carry

            return lax.cond(bmax > t, lambda: slow(carry), lambda: carry)

        cand_v, cand_i, _ = lax.fori_loop(
            0, NBLK, blk_body,
            (jnp.full((L,), NEG, jnp.float32), jnp.zeros((L,), jnp.int32),
             jnp.float32(NEG)))

        bits = plsc.load_gather(tgt_v, [cand_i]).astype(jnp.float32)
        hits = bits * mask12
        tp = plsc.cumsum(hits)
        metric = jnp.sum(tp * inv_ranks * hits)
        denom_i = jnp.minimum(jnp.int32(K), tsum)
        recip = jnp.max(jnp.where(iota == denom_i, rec_denom, NEG))
        ap = metric * recip

        res_v[...] = jnp.where(iota == 0, ap, 0.0)
        pltpu.sync_copy(res_v, out_hbm.at[row])


@jax.jit
def _sc_map(logits, targets):
    mesh = plsc.VectorSubcoreMesh(core_axis_name="c", subcore_axis_name="s")
    f = pl.kernel(
        _sc_body,
        out_type=jax.ShapeDtypeStruct((B, L), jnp.float32),
        mesh=mesh,
        scratch_types=[
            pltpu.VMEM((N,), jnp.float32),
            pltpu.VMEM((N,), jnp.int32),
            pltpu.VMEM((2, L), jnp.float32),
            pltpu.VMEM((L,), jnp.float32),
        ],
        compiler_params=pltpu.CompilerParams(needs_layout_passes=False),
    )
    inv_ranks = 1.0 / (jnp.arange(L, dtype=jnp.float32) + 1.0)
    rec = jnp.arange(L, dtype=jnp.float32)
    rec_denom = jnp.where((rec >= 1) & (rec <= K), 1.0 / jnp.maximum(rec, 1.0),
                          jnp.where(rec == 0, jnp.inf, 0.0))
    tab = jnp.stack([inv_ranks, rec_denom]).astype(jnp.float32)
    return f(logits, targets, tab)


def kernel(logits, targets):
    return jnp.sum(_sc_map(logits, targets))


# speedup vs baseline: 1.8537x; 1.1035x over previous
import jax
import jax.numpy as jnp
from jax import lax
from jax.experimental import pallas as pl
from jax.experimental.pallas import tpu as pltpu
from jax.experimental.pallas import tpu_sc as plsc

B = 128
N = 32768
K = 12
L = 16
NW = 32
ROWS_PER_W = B // NW
BLOCK = 1024
SUB = 128
NEG = -3.0e38


def _lane(x, k):
    i = lax.iota(jnp.int32, L)
    return jnp.max(jnp.where(i == k, x, NEG))


def _tree_max(vs):
    while len(vs) > 1:
        vs = [jnp.maximum(vs[i], vs[i + 1]) for i in range(0, len(vs) - 1, 2)] \
            + ([vs[-1]] if len(vs) % 2 else [])
    return vs[0]


def _merge(v, base, cand_v, cand_i):
    idx = base + lax.iota(jnp.int32, L)
    sv, si = plsc.sort_key_val(v, idx, descending=True)
    rcv = lax.rev(cand_v, (0,))
    rci = lax.rev(cand_i, (0,))
    take = (sv > rcv) | ((sv == rcv) & (si < rci))
    nv = jnp.where(take, sv, rcv)
    ni = jnp.where(take, si, rci)
    cv2, ci2 = plsc.sort_key_val(nv, ni, descending=True)
    return cv2, ci2, _lane(cv2, K - 1)


def _tie_fixup(cand_v, cand_i, fv_ref, fi_ref, iota):
    for phase in range(2):
        if phase == 0:
            partner = jnp.bitwise_xor(iota, 1)
        else:
            partner = jnp.where((iota >= 1) & (iota <= 14),
                                jnp.bitwise_xor(iota - 1, 1) + 1, iota)
        fv_ref[...] = cand_v
        fi_ref[...] = cand_i
        pv = plsc.load_gather(fv_ref, [partner])
        pi = plsc.load_gather(fi_ref, [partner])
        win = (cand_v > pv) | ((cand_v == pv) & (cand_i < pi))
        lower = iota < partner
        keep = (lower & win) | (~lower & ~win)
        cand_v = jnp.where(keep, cand_v, pv)
        cand_i = jnp.where(keep, cand_i, pi)
    return cand_v, cand_i


def _sc_body(logits_hbm, targets_hbm, tab_hbm, out_hbm,
             log_a, log_b, tgt_v, tab_v, res_v, fv_ref, fi_ref,
             sem_l, sem_t, sem_o):
    wid = lax.axis_index("c") * 16 + lax.axis_index("s")
    iota = lax.iota(jnp.int32, L)
    pltpu.sync_copy(tab_hbm, tab_v)
    inv_ranks = tab_v[0]
    rec_denom = tab_v[1]
    mask12 = (iota < K).astype(jnp.float32)

    r0 = wid * ROWS_PER_W
    logbufs = [log_a, log_b]
    h_log = pltpu.async_copy(logits_hbm.at[r0], log_a, sem_l)
    h_tgt = pltpu.async_copy(targets_hbm.at[r0], tgt_v, sem_t)
    out_handles = []

    for k in range(ROWS_PER_W):
        row = r0 + k
        log_v = logbufs[k % 2]
        h_log.wait()
        if k + 1 < ROWS_PER_W:
            h_log = pltpu.async_copy(
                logits_hbm.at[row + 1], logbufs[(k + 1) % 2], sem_l)

        def blk_body(b, carry, log_v=log_v):
            cand_v, cand_i, t = carry
            base = b * BLOCK
            accs = [log_v[pl.ds(base + u * L, L)] for u in range(8)]
            for j in range(8, BLOCK // L):
                accs[j % 8] = jnp.maximum(
                    accs[j % 8], log_v[pl.ds(base + j * L, L)])
            bmax = jnp.max(_tree_max(accs))

            def slow(carry):
                def sb_body(sb, carry):
                    sbase = base + sb * SUB
                    cs = [log_v[pl.ds(sbase + u * L, L)] for u in range(8)]
                    sbmax = jnp.max(_tree_max(cs))

                    def sb_slow(carry):
                        def ch_body(u, carry):
                            cv, ci, tt = carry
                            cbase = sbase + u * L
                            v = log_v[pl.ds(cbase, L)]
                            cmax = jnp.max(v)
                            return lax.cond(
                                cmax > tt,
                                lambda: _merge(v, cbase, cv, ci),
                                lambda: (cv, ci, tt))
                        return lax.fori_loop(0, SUB // L, ch_body, carry)

                    tt = carry[2]
                    return lax.cond(sbmax > tt, lambda: sb_slow(carry),
                                    lambda: carry)
                return lax.fori_loop(0, BLOCK // SUB, sb_body, carry)

            return lax.cond(bmax > t, lambda: slow(carry), lambda: carry)

        cand_v, cand_i, _ = lax.fori_loop(
            0, N // BLOCK, blk_body,
            (jnp.full((L,), NEG, jnp.float32), jnp.zeros((L,), jnp.int32),
             jnp.float32(NEG)))

        h_tgt.wait()

        def sum_body(b, acc):
            base = b * (L * 16)
            for u in range(16):
                acc = acc + tgt_v[pl.ds(base + u * L, L)]
            return acc

        acc = lax.fori_loop(0, N // (L * 16), sum_body,
                            jnp.zeros((L,), jnp.int32))
        tsum = jnp.sum(acc)

        cand_v, cand_i = _tie_fixup(cand_v, cand_i, fv_ref, fi_ref, iota)
        bits = plsc.load_gather(tgt_v, [cand_i]).astype(jnp.float32)
        hits = bits * mask12
        tp = plsc.cumsum(hits)
        metric = jnp.sum(tp * inv_ranks * hits)
        denom_i = jnp.minimum(jnp.int32(K), tsum)
        recip = jnp.max(jnp.where(iota == denom_i, rec_denom, NEG))
        ap = metric * recip

        res_v[k, :] = jnp.where(iota == 0, ap, 0.0)
        out_handles.append(
            pltpu.async_copy(res_v.at[k], out_hbm.at[row], sem_o))
        if k + 1 < ROWS_PER_W:
            h_tgt = pltpu.async_copy(targets_hbm.at[row + 1], tgt_v, sem_t)

    for h in out_handles:
        h.wait()


@jax.jit
def _sc_map(logits, targets):
    mesh = plsc.VectorSubcoreMesh(core_axis_name="c", subcore_axis_name="s")
    f = pl.kernel(
        _sc_body,
        out_type=jax.ShapeDtypeStruct((B, L), jnp.float32),
        mesh=mesh,
        scratch_types=[
            pltpu.VMEM((N,), jnp.float32),
            pltpu.VMEM((N,), jnp.float32),
            pltpu.VMEM((N,), jnp.int32),
            pltpu.VMEM((2, L), jnp.float32),
            pltpu.VMEM((ROWS_PER_W, L), jnp.float32),
            pltpu.VMEM((L,), jnp.float32),
            pltpu.VMEM((L,), jnp.int32),
            pltpu.SemaphoreType.DMA,
            pltpu.SemaphoreType.DMA,
            pltpu.SemaphoreType.DMA,
        ],
        compiler_params=pltpu.CompilerParams(needs_layout_passes=False),
    )
    inv_ranks = 1.0 / (jnp.arange(L, dtype=jnp.float32) + 1.0)
    rec = jnp.arange(L, dtype=jnp.float32)
    rec_denom = jnp.where((rec >= 1) & (rec <= K), 1.0 / jnp.maximum(rec, 1.0),
                          jnp.where(rec == 0, jnp.inf, 0.0))
    tab = jnp.stack([inv_ranks, rec_denom]).astype(jnp.float32)
    return f(logits, targets, tab)


def kernel(logits, targets):
    return jnp.sum(_sc_map(logits, targets))


# speedup vs baseline: 2.8941x; 1.5612x over previous
import jax
import jax.numpy as jnp
from jax import lax
from jax.experimental import pallas as pl
from jax.experimental.pallas import tpu as pltpu
from jax.experimental.pallas import tpu_sc as plsc

B = 128
N = 32768
K = 12
L = 16
NW = 32
ROWS_PER_W = B // NW
BLOCK = 1024
SUB = 128
NEG = -3.0e38


def _lane(x, k):
    i = lax.iota(jnp.int32, L)
    return jnp.max(jnp.where(i == k, x, NEG))


def _tree_max(vs):
    while len(vs) > 1:
        vs = [jnp.maximum(vs[i], vs[i + 1]) for i in range(0, len(vs) - 1, 2)] \
            + ([vs[-1]] if len(vs) % 2 else [])
    return vs[0]


def _merge16(av, ai, bv, bi):
    rbv = lax.rev(bv, (0,))
    rbi = lax.rev(bi, (0,))
    take = (av > rbv) | ((av == rbv) & (ai < rbi))
    nv = jnp.where(take, av, rbv)
    ni = jnp.where(take, ai, rbi)
    return plsc.sort_key_val(nv, ni, descending=True)


def _sub_top16(log_v, sbase):
    iota = lax.iota(jnp.int32, L)
    pairs = []
    for u in range(8):
        v = log_v[pl.ds(sbase + u * L, L)]
        idx = sbase + u * L + iota
        pairs.append(plsc.sort_key_val(v, idx, descending=True))
    while len(pairs) > 1:
        pairs = [_merge16(*pairs[i], *pairs[i + 1])
                 for i in range(0, len(pairs), 2)]
    return pairs[0]


def _tie_fixup(cand_v, cand_i, fv_ref, fi_ref, iota):
    for phase in range(2):
        if phase == 0:
            partner = jnp.bitwise_xor(iota, 1)
        else:
            partner = jnp.where((iota >= 1) & (iota <= 14),
                                jnp.bitwise_xor(iota - 1, 1) + 1, iota)
        fv_ref[...] = cand_v
        fi_ref[...] = cand_i
        pv = plsc.load_gather(fv_ref, [partner])
        pi = plsc.load_gather(fi_ref, [partner])
        win = (cand_v > pv) | ((cand_v == pv) & (cand_i < pi))
        lower = iota < partner
        keep = (lower & win) | (~lower & ~win)
        cand_v = jnp.where(keep, cand_v, pv)
        cand_i = jnp.where(keep, cand_i, pi)
    return cand_v, cand_i


def _sc_body(logits_hbm, targets_hbm, tab_hbm, out_hbm,
             log_a, log_b, tgt_v, tab_v, res_v, fv_ref, fi_ref,
             sem_l, sem_t, sem_o):
    wid = lax.axis_index("c") * 16 + lax.axis_index("s")
    iota = lax.iota(jnp.int32, L)
    pltpu.sync_copy(tab_hbm, tab_v)
    inv_ranks = tab_v[0]
    rec_denom = tab_v[1]
    mask12 = (iota < K).astype(jnp.float32)

    r0 = wid * ROWS_PER_W
    logbufs = [log_a, log_b]
    h_log = pltpu.async_copy(logits_hbm.at[r0], log_a, sem_l)
    h_tgt = pltpu.async_copy(targets_hbm.at[r0], tgt_v, sem_t)
    out_handles = []

    for k in range(ROWS_PER_W):
        row = r0 + k
        log_v = logbufs[k % 2]
        h_log.wait()
        if k + 1 < ROWS_PER_W:
            h_log = pltpu.async_copy(
                logits_hbm.at[row + 1], logbufs[(k + 1) % 2], sem_l)

        def blk_body(b, carry, log_v=log_v):
            cand_v, cand_i, t = carry
            base = b * BLOCK
            accs = [log_v[pl.ds(base + u * L, L)] for u in range(8)]
            for j in range(8, BLOCK // L):
                accs[j % 8] = jnp.maximum(
                    accs[j % 8], log_v[pl.ds(base + j * L, L)])
            bmax = jnp.max(_tree_max(accs))

            def slow(carry):
                def sb_body(sb, carry):
                    sbase = base + sb * SUB
                    cs = [log_v[pl.ds(sbase + u * L, L)] for u in range(8)]
                    sbmax = jnp.max(_tree_max(cs))

                    def sb_slow(carry):
                        cv, ci, _ = carry
                        sv, si = _sub_top16(log_v, sbase)
                        cv2, ci2 = _merge16(sv, si, cv, ci)
                        return cv2, ci2, _lane(cv2, K - 1)

                    tt = carry[2]
                    return lax.cond(sbmax > tt, lambda: sb_slow(carry),
                                    lambda: carry)
                return lax.fori_loop(0, BLOCK // SUB, sb_body, carry)

            return lax.cond(bmax > t, lambda: slow(carry), lambda: carry)

        cand_v, cand_i, _ = lax.fori_loop(
            0, N // BLOCK, blk_body,
            (jnp.full((L,), NEG, jnp.float32), jnp.zeros((L,), jnp.int32),
             jnp.float32(NEG)))

        h_tgt.wait()

        def sum_body(b, acc):
            base = b * (L * 16)
            for u in range(16):
                acc = acc + tgt_v[pl.ds(base + u * L, L)]
            return acc

        acc = lax.fori_loop(0, N // (L * 16), sum_body,
                            jnp.zeros((L,), jnp.int32))
        tsum = jnp.sum(acc)

        cand_v, cand_i = _tie_fixup(cand_v, cand_i, fv_ref, fi_ref, iota)
        bits = plsc.load_gather(tgt_v, [cand_i]).astype(jnp.float32)
        hits = bits * mask12
        tp = plsc.cumsum(hits)
        metric = jnp.sum(tp * inv_ranks * hits)
        denom_i = jnp.minimum(jnp.int32(K), tsum)
        recip = jnp.max(jnp.where(iota == denom_i, rec_denom, NEG))
        ap = metric * recip

        res_v[k, :] = jnp.where(iota == 0, ap, 0.0)
        out_handles.append(
            pltpu.async_copy(res_v.at[k], out_hbm.at[row], sem_o))
        if k + 1 < ROWS_PER_W:
            h_tgt = pltpu.async_copy(targets_hbm.at[row + 1], tgt_v, sem_t)

    for h in out_handles:
        h.wait()


@jax.jit
def _sc_map(logits, targets):
    mesh = plsc.VectorSubcoreMesh(core_axis_name="c", subcore_axis_name="s")
    f = pl.kernel(
        _sc_body,
        out_type=jax.ShapeDtypeStruct((B, L), jnp.float32),
        mesh=mesh,
        scratch_types=[
            pltpu.VMEM((N,), jnp.float32),
            pltpu.VMEM((N,), jnp.float32),
            pltpu.VMEM((N,), jnp.int32),
            pltpu.VMEM((2, L), jnp.float32),
            pltpu.VMEM((ROWS_PER_W, L), jnp.float32),
            pltpu.VMEM((L,), jnp.float32),
            pltpu.VMEM((L,), jnp.int32),
            pltpu.SemaphoreType.DMA,
            pltpu.SemaphoreType.DMA,
            pltpu.SemaphoreType.DMA,
        ],
        compiler_params=pltpu.CompilerParams(needs_layout_passes=False),
    )
    inv_ranks = 1.0 / (jnp.arange(L, dtype=jnp.float32) + 1.0)
    rec = jnp.arange(L, dtype=jnp.float32)
    rec_denom = jnp.where((rec >= 1) & (rec <= K), 1.0 / jnp.maximum(rec, 1.0),
                          jnp.where(rec == 0, jnp.inf, 0.0))
    tab = jnp.stack([inv_ranks, rec_denom]).astype(jnp.float32)
    return f(logits, targets, tab)


def kernel(logits, targets):
    return jnp.sum(_sc_map(logits, targets))
